# baseline (speedup 1.0000x reference)
import functools

import jax
import jax.numpy as jnp
from jax import lax
from jax.experimental import pallas as pl
from jax.experimental.pallas import tpu as pltpu
from jax.experimental.pallas import tpu_sc as plsc

NSTATES = 100000
NACT = 128
NBATCH = 16384
LRATE = 0.1
DISCOUNT = 0.99

NC = 2
NS = 16
L = 16
NWORK = NC * NS
BPW = NBATCH // NWORK
GCHUNK = 128
WIN_R = 128
WKEYS = WIN_R * NACT
NWIN_TOT = -(-NSTATES // WIN_R)
LAST_WIN = NWIN_TOT - 1
LAST_R = NSTATES - LAST_WIN * WIN_R
SLOTS = -(-NWIN_TOT // NWORK)
KCH = 1024
CAP = NBATCH + L
NFLAG = SLOTS * WIN_R
RCH = 128

_mesh = plsc.VectorSubcoreMesh(
    core_axis_name="c", subcore_axis_name="s", num_cores=NC, num_subcores=NS
)
_params = pltpu.CompilerParams(needs_layout_passes=False)


def _worker_id():
    return lax.axis_index("s") * NC + lax.axis_index("c")


def _targets_body(q_hbm, ns_hbm, st_hbm, ac_hbm, rw_hbm, dn_hbm,
                  key_hbm, tgt_hbm,
                  ns_v, st_v, ac_v, rw_v, dn_v, rows_v, cmax_v, key_v, tgt_v,
                  sem):
    base = _worker_id() * BPW
    pltpu.sync_copy(ns_hbm.at[pl.ds(base, BPW)], ns_v)
    pltpu.sync_copy(st_hbm.at[pl.ds(base, BPW)], st_v)
    pltpu.sync_copy(ac_hbm.at[pl.ds(base, BPW)], ac_v)
    pltpu.sync_copy(rw_hbm.at[pl.ds(base, BPW)], rw_v)
    pltpu.sync_copy(dn_hbm.at[pl.ds(base, BPW)], dn_v)
    iota = lax.iota(jnp.int32, L)
    for ci in range(BPW // GCHUNK):
        pltpu.async_copy(
            q_hbm.at[ns_v.at[pl.ds(ci * GCHUNK, GCHUNK)]], rows_v, sem
        ).wait()

        def row_body(r, carry):
            acc = rows_v[r, pl.ds(0, L)]
            for k in range(1, NACT // L):
                acc = jnp.maximum(acc, rows_v[r, pl.ds(k * L, L)])
            cmax_v[pl.ds(r * (L + 1), L)] = acc
            return carry

        lax.fori_loop(0, GCHUNK, row_body, 0)
        for g in range(GCHUNK // L):
            ridx = (g * L + iota) * (L + 1)

            def col_body(j, m):
                jj = jnp.broadcast_to(j, (L,)).astype(jnp.int32)
                return jnp.maximum(m, plsc.load_gather(cmax_v, [ridx + jj]))

            m0 = plsc.load_gather(cmax_v, [ridx])
            m = lax.fori_loop(1, L, col_body, m0)
            off = ci * GCHUNK + g * L
            rw = rw_v[pl.ds(off, L)]
            dn = dn_v[pl.ds(off, L)]
            st = st_v[pl.ds(off, L)]
            ac = ac_v[pl.ds(off, L)]
            tgt_v[pl.ds(off, L)] = rw + DISCOUNT * m * (1.0 - dn)
            key_v[pl.ds(off, L)] = st * NACT + ac
    pltpu.sync_copy(key_v, key_hbm.at[pl.ds(base, BPW)])
    pltpu.sync_copy(tgt_v, tgt_hbm.at[pl.ds(base, BPW)])


_targets_call = functools.partial(
    pl.kernel,
    out_type=(
        jax.ShapeDtypeStruct((NBATCH,), jnp.int32),
        jax.ShapeDtypeStruct((NBATCH,), jnp.float32),
    ),
    mesh=_mesh,
    scratch_types=[
        pltpu.VMEM((BPW,), jnp.int32),
        pltpu.VMEM((BPW,), jnp.int32),
        pltpu.VMEM((BPW,), jnp.int32),
        pltpu.VMEM((BPW,), jnp.float32),
        pltpu.VMEM((BPW,), jnp.float32),
        pltpu.VMEM((GCHUNK, NACT), jnp.float32),
        pltpu.VMEM((GCHUNK * (L + 1),), jnp.float32),
        pltpu.VMEM((BPW,), jnp.int32),
        pltpu.VMEM((BPW,), jnp.float32),
        pltpu.SemaphoreType.DMA,
    ],
    compiler_params=_params,
    name="q_targets",
)(_targets_body)


def _apply_body(q_hbm, key_hbm, tgt_hbm, out_hbm,
                kch0_v, kch1_v, tch0_v, tch1_v, lkey_v, ltgt_v,
                flags_v, ulist_v, fidx_v, rows_v, widx_v, wc_v,
                ksem0, ksem1, gsem, ssem, csem):
    kchs = (kch0_v, kch1_v)
    tchs = (tch0_v, tch1_v)
    ksems = (ksem0, ksem1)
    wid = _worker_id()
    iota = lax.iota(jnp.int32, L)

    def slot_win(s):
        return wid + s * NWORK

    def copy_desc(s):
        row0 = pl.multiple_of(slot_win(s) * WIN_R, 8)
        return pltpu.make_async_copy(q_hbm.at[pl.ds(row0, WIN_R)],
                                     out_hbm.at[pl.ds(row0, WIN_R)], csem)

    def copy_last_desc():
        row0 = LAST_WIN * WIN_R
        return pltpu.make_async_copy(q_hbm.at[pl.ds(row0, LAST_R)],
                                     out_hbm.at[pl.ds(row0, LAST_R)], csem)

    def each_copy(fn_full, fn_last):
        for s in range(SLOTS):
            @pl.when(slot_win(s) < LAST_WIN)
            def _(s=s):
                fn_full(s)

            @pl.when(slot_win(s) == LAST_WIN)
            def _():
                fn_last()

    each_copy(lambda s: copy_desc(s).start(),
              lambda: copy_last_desc().start())

    def kch_copies(ch, b):
        src_k = key_hbm.at[pl.ds(ch * KCH, KCH)]
        src_t = tgt_hbm.at[pl.ds(ch * KCH, KCH)]
        return (pltpu.make_async_copy(src_k, kchs[b], ksems[b]),
                pltpu.make_async_copy(src_t, tchs[b], ksems[b]))

    def make_filt(b):
        def filt(i, n):
            k = kchs[b][pl.ds(i * L, L)]
            t = tchs[b][pl.ds(i * L, L)]
            m = ((k >> 14) & (NWORK - 1)) == wid
            plsc.store_compressed(lkey_v.at[pl.ds(n, L)], k, mask=m)
            plsc.store_compressed(ltgt_v.at[pl.ds(n, L)], t, mask=m)
            return n + plsc.all_reduce_population_count(m)[0]
        return filt

    NKCH = NBATCH // KCH
    for d in kch_copies(0, 0):
        d.start()
    n_loc = jnp.int32(0)
    for ch in range(NKCH):
        b = ch % 2
        if ch + 1 < NKCH:
            for d in kch_copies(ch + 1, 1 - b):
                d.start()
        for d in kch_copies(ch, b):
            d.wait()
        n_loc = lax.fori_loop(0, KCH // L, make_filt(b), n_loc)
    nvec = (n_loc + (L - 1)) // L

    def set_flags(i, carry):
        k = lkey_v[pl.ds(i * L, L)]
        mv = (i * L + iota) < n_loc
        row = k >> 7
        fi = jnp.where(mv, ((row >> 12) << 7) | (row & (WIN_R - 1)), 0)
        plsc.store_scatter(flags_v, [fi], jnp.ones((L,), jnp.int32), mask=mv)
        return carry

    def zero_flags(i, carry):
        flags_v[pl.ds(i * L, L)] = jnp.zeros((L,), jnp.int32)
        return carry

    lax.fori_loop(0, NFLAG // L, zero_flags, 0)
    lax.fori_loop(0, nvec, set_flags, 0)

    def compact(i, n):
        f = flags_v[pl.ds(i * L, L)]
        m = f > 0
        fi = i * L + iota
        rowg = (wid << 7) + ((fi >> 7) << 12) + (fi & (WIN_R - 1))
        pos = n + plsc.cumsum(m.astype(jnp.int32)) - 1
        plsc.store_compressed(ulist_v.at[pl.ds(n, L)], rowg, mask=m)
        plsc.store_scatter(flags_v, [fi], pos, mask=m)
        return n + plsc.all_reduce_population_count(m)[0]

    n_rows = lax.fori_loop(0, NFLAG // L, compact, jnp.int32(0))
    n_chunks = (n_rows + (RCH - 1)) // RCH
    last_i = jnp.maximum(n_rows - 1, 0)
    last_row = jnp.broadcast_to(ulist_v[pl.ds(last_i, L)][0], (L,))

    each_copy(lambda s: copy_desc(s).wait(),
              lambda: copy_last_desc().wait())

    for c in range(NFLAG // RCH):
        start = c * RCH

        @pl.when(jnp.int32(c) < n_chunks)
        def _(c=c, start=start):
            valid = jnp.minimum(jnp.int32(RCH), n_rows - start)
            for k8 in range(RCH // L):
                v = ulist_v[pl.ds(start + k8 * L, L)]
                lane_pos = k8 * L + iota
                fidx_v[pl.ds(k8 * L, L)] = jnp.where(lane_pos < valid,
                                                     v, last_row)
            pltpu.async_copy(q_hbm.at[fidx_v], rows_v, gsem).wait()

            def conv(i, nw):
                k = lkey_v[pl.ds(i * L, L)]
                t = ltgt_v[pl.ds(i * L, L)]
                mv = (i * L + iota) < n_loc
                row = k >> 7
                fi = jnp.where(mv, ((row >> 12) << 7) | (row & (WIN_R - 1)),
                               0)
                s = plsc.load_gather(flags_v, [fi], mask=mv)
                m2 = mv & (s >= start) & (s < start + RCH)
                lrow = jnp.where(m2, s - start, 0)
                colv = k & (NACT - 1)
                q = plsc.load_gather(rows_v, [lrow, colv], mask=m2)
                plsc.store_compressed(widx_v.at[pl.ds(nw, L)],
                                      (lrow << 7) | colv, mask=m2)
                plsc.store_compressed(wc_v.at[pl.ds(nw, L)],
                                      LRATE * (t - q), mask=m2)
                return nw + plsc.all_reduce_population_count(m2)[0]

            nw = lax.fori_loop(0, nvec, conv, jnp.int32(0))

            def apply(i, carry):
                li = widx_v[pl.ds(i * L, L)]
                cv = wc_v[pl.ds(i * L, L)]
                lv = (i * L + iota) < nw
                lrow = li >> 7
                colv = li & (NACT - 1)
                for lane in range(L):
                    plsc.addupdate_scatter(rows_v, [lrow, colv], cv,
                                           mask=lv & (iota == lane))
                return carry

            lax.fori_loop(0, (nw + (L - 1)) // L, apply, 0)

            @pl.when(valid < RCH)
            def _():
                def padrow(p, carry):
                    for k8 in range(NACT // L):
                        rows_v[p, pl.ds(k8 * L, L)] = (
                            rows_v[valid - 1, pl.ds(k8 * L, L)])
                    return carry

                lax.fori_loop(valid, RCH, padrow, 0)

            pltpu.async_copy(rows_v, out_hbm.at[fidx_v], ssem).wait()


_apply_call = functools.partial(
    pl.kernel,
    out_type=jax.ShapeDtypeStruct((NSTATES, NACT), jnp.float32),
    mesh=_mesh,
    scratch_types=[
        pltpu.VMEM((KCH,), jnp.int32),
        pltpu.VMEM((KCH,), jnp.int32),
        pltpu.VMEM((KCH,), jnp.float32),
        pltpu.VMEM((KCH,), jnp.float32),
        pltpu.VMEM((CAP,), jnp.int32),
        pltpu.VMEM((CAP,), jnp.float32),
        pltpu.VMEM((NFLAG,), jnp.int32),
        pltpu.VMEM((NFLAG + L,), jnp.int32),
        pltpu.VMEM((RCH,), jnp.int32),
        pltpu.VMEM((RCH, NACT), jnp.float32),
        pltpu.VMEM((CAP,), jnp.int32),
        pltpu.VMEM((CAP,), jnp.float32),
        pltpu.SemaphoreType.DMA,
        pltpu.SemaphoreType.DMA,
        pltpu.SemaphoreType.DMA,
        pltpu.SemaphoreType.DMA,
        pltpu.SemaphoreType.DMA,
    ],
    compiler_params=_params,
    name="q_apply",
)(_apply_body)


def kernel(q_table, states, actions, rewards, next_states, dones):
    states = states.astype(jnp.int32)
    actions = actions.astype(jnp.int32)
    next_states = next_states.astype(jnp.int32)
    dones_f = dones.astype(jnp.float32)
    keys, tgts = _targets_call(q_table, next_states, states, actions,
                               rewards, dones_f)
    return _apply_call(q_table, keys, tgts)


# speedup vs baseline: 16.8241x; 16.8241x over previous
import functools

import jax
import jax.numpy as jnp
from jax import lax
from jax.experimental import pallas as pl
from jax.experimental.pallas import tpu as pltpu
from jax.experimental.pallas import tpu_sc as plsc

NSTATES = 100000
NACT = 128
NBATCH = 16384
LRATE = 0.1
DISCOUNT = 0.99

NC = 2
NS = 16
L = 16
NWORK = NC * NS
BPW = NBATCH // NWORK
GCHUNK = 128
WIN_R = 128
WKEYS = WIN_R * NACT
NWIN_TOT = -(-NSTATES // WIN_R)
LAST_WIN = NWIN_TOT - 1
LAST_R = NSTATES - LAST_WIN * WIN_R
SLOTS = -(-NWIN_TOT // NWORK)
KCH = 1024
CAP = NBATCH + L

_mesh = plsc.VectorSubcoreMesh(
    core_axis_name="c", subcore_axis_name="s", num_cores=NC, num_subcores=NS
)
_params = pltpu.CompilerParams(needs_layout_passes=False)


def _worker_id():
    return lax.axis_index("s") * NC + lax.axis_index("c")


def _targets_body(q_hbm, ns_hbm, st_hbm, ac_hbm, rw_hbm, dn_hbm,
                  key_hbm, tgt_hbm,
                  ns_v, st_v, ac_v, rw_v, dn_v, rows0_v, rows1_v, cmax_v,
                  key_v, tgt_v, sem0, sem1):
    rows = (rows0_v, rows1_v)
    sems = (sem0, sem1)
    base = _worker_id() * BPW
    pltpu.sync_copy(ns_hbm.at[pl.ds(base, BPW)], ns_v)
    pltpu.sync_copy(st_hbm.at[pl.ds(base, BPW)], st_v)
    pltpu.sync_copy(ac_hbm.at[pl.ds(base, BPW)], ac_v)
    pltpu.sync_copy(rw_hbm.at[pl.ds(base, BPW)], rw_v)
    pltpu.sync_copy(dn_hbm.at[pl.ds(base, BPW)], dn_v)
    iota = lax.iota(jnp.int32, L)

    def gather_desc(ci, b):
        return pltpu.make_async_copy(
            q_hbm.at[ns_v.at[pl.ds(ci * GCHUNK, GCHUNK)]], rows[b], sems[b])

    NCI = BPW // GCHUNK
    gather_desc(0, 0).start()
    for ci in range(NCI):
        b = ci % 2
        if ci + 1 < NCI:
            gather_desc(ci + 1, 1 - b).start()
        gather_desc(ci, b).wait()
        rows_v = rows[b]

        def row_body(r, carry):
            acc = rows_v[r, pl.ds(0, L)]
            for k in range(1, NACT // L):
                acc = jnp.maximum(acc, rows_v[r, pl.ds(k * L, L)])
            cmax_v[pl.ds(r * (L + 1), L)] = acc
            return carry

        lax.fori_loop(0, GCHUNK, row_body, 0)
        for g in range(GCHUNK // L):
            ridx = (g * L + iota) * (L + 1)

            def col_body(j, m):
                jj = jnp.broadcast_to(j, (L,)).astype(jnp.int32)
                return jnp.maximum(m, plsc.load_gather(cmax_v, [ridx + jj]))

            m0 = plsc.load_gather(cmax_v, [ridx])
            m = lax.fori_loop(1, L, col_body, m0)
            off = ci * GCHUNK + g * L
            rw = rw_v[pl.ds(off, L)]
            dn = dn_v[pl.ds(off, L)]
            st = st_v[pl.ds(off, L)]
            ac = ac_v[pl.ds(off, L)]
            tgt_v[pl.ds(off, L)] = rw + DISCOUNT * m * (1.0 - dn)
            key_v[pl.ds(off, L)] = st * NACT + ac
    pltpu.sync_copy(key_v, key_hbm.at[pl.ds(base, BPW)])
    pltpu.sync_copy(tgt_v, tgt_hbm.at[pl.ds(base, BPW)])


_targets_call = functools.partial(
    pl.kernel,
    out_type=(
        jax.ShapeDtypeStruct((NBATCH,), jnp.int32),
        jax.ShapeDtypeStruct((NBATCH,), jnp.float32),
    ),
    mesh=_mesh,
    scratch_types=[
        pltpu.VMEM((BPW,), jnp.int32),
        pltpu.VMEM((BPW,), jnp.int32),
        pltpu.VMEM((BPW,), jnp.int32),
        pltpu.VMEM((BPW,), jnp.float32),
        pltpu.VMEM((BPW,), jnp.float32),
        pltpu.VMEM((GCHUNK, NACT), jnp.float32),
        pltpu.VMEM((GCHUNK, NACT), jnp.float32),
        pltpu.VMEM((GCHUNK * (L + 1),), jnp.float32),
        pltpu.VMEM((BPW,), jnp.int32),
        pltpu.VMEM((BPW,), jnp.float32),
        pltpu.SemaphoreType.DMA,
        pltpu.SemaphoreType.DMA,
    ],
    compiler_params=_params,
    name="q_targets",
)(_targets_body)


def _apply_body(q_hbm, key_hbm, tgt_hbm, out_hbm,
                kch0_v, kch1_v, tch0_v, tch1_v, lkey_v, ltgt_v,
                win0_v, win1_v, win2_v, wlast_v, widx_v, wc_v,
                ksem0, ksem1, lsem0, lsem1, lsem2, ssem0, ssem1, ssem2):
    kchs = (kch0_v, kch1_v)
    tchs = (tch0_v, tch1_v)
    wins = (win0_v, win1_v, win2_v)
    lsems = (lsem0, lsem1, lsem2)
    ssems = (ssem0, ssem1, ssem2)
    ksems = (ksem0, ksem1)
    wid = _worker_id()
    iota = lax.iota(jnp.int32, L)

    def kch_copies(ch, b):
        src_k = key_hbm.at[pl.ds(ch * KCH, KCH)]
        src_t = tgt_hbm.at[pl.ds(ch * KCH, KCH)]
        return (pltpu.make_async_copy(src_k, kchs[b], ksems[b]),
                pltpu.make_async_copy(src_t, tchs[b], ksems[b]))

    def make_filt(b):
        def filt(i, n):
            k = kchs[b][pl.ds(i * L, L)]
            t = tchs[b][pl.ds(i * L, L)]
            m = ((k >> 14) & (NWORK - 1)) == wid
            plsc.store_compressed(lkey_v.at[pl.ds(n, L)], k, mask=m)
            plsc.store_compressed(ltgt_v.at[pl.ds(n, L)], t, mask=m)
            return n + plsc.all_reduce_population_count(m)[0]
        return filt

    def slot_win(s):
        return wid + s * NWORK

    def load_desc(s, b):
        row0 = pl.multiple_of(slot_win(s) * WIN_R, 8)
        return pltpu.make_async_copy(q_hbm.at[pl.ds(row0, WIN_R)],
                                     wins[b], lsems[b])

    def store_desc(s, b):
        row0 = pl.multiple_of(slot_win(s) * WIN_R, 8)
        return pltpu.make_async_copy(wins[b],
                                     out_hbm.at[pl.ds(row0, WIN_R)], ssems[b])

    NKCH = NBATCH // KCH
    for d in kch_copies(0, 0):
        d.start()
    for s in range(3):
        @pl.when(slot_win(s) < LAST_WIN)
        def _(s=s):
            load_desc(s, s).start()
    n_loc = jnp.int32(0)
    for ch in range(NKCH):
        b = ch % 2
        if ch + 1 < NKCH:
            for d in kch_copies(ch + 1, 1 - b):
                d.start()
        for d in kch_copies(ch, b):
            d.wait()
        n_loc = lax.fori_loop(0, KCH // L, make_filt(b), n_loc)
    nvec = (n_loc + (L - 1)) // L

    def _process(win, win_v):
        def collect(i, nw):
            k = lkey_v[pl.ds(i * L, L)]
            t = ltgt_v[pl.ds(i * L, L)]
            lane = i * L + iota
            m = (lane < n_loc) & ((k >> 14) == win)
            li = jnp.where(m, k & (WKEYS - 1), 0)
            q = plsc.load_gather(win_v, [li >> 7, li & (NACT - 1)], mask=m)
            c = LRATE * (t - q)
            plsc.store_compressed(widx_v.at[pl.ds(nw, L)], li, mask=m)
            plsc.store_compressed(wc_v.at[pl.ds(nw, L)], c, mask=m)
            return nw + plsc.all_reduce_population_count(m)[0]

        nw = lax.fori_loop(0, nvec, collect, jnp.int32(0))

        def apply_blk(i, carry):
            li = widx_v[pl.ds(i * L, L)]
            cv = wc_v[pl.ds(i * L, L)]
            lane_valid = i * L + iota < nw
            rv = li >> 7
            colv = li & (NACT - 1)
            for lane in range(L):
                m = (iota == lane) & lane_valid
                plsc.addupdate_scatter(win_v, [rv, colv], cv, mask=m)
            return carry

        lax.fori_loop(0, (nw + (L - 1)) // L, apply_blk, 0)

    for s in range(SLOTS):
        b = s % 3
        u = s + 1
        if 3 <= u < SLOTS:
            @pl.when(slot_win(u) < LAST_WIN)
            def _(s=s, u=u):
                store_desc(u - 3, u % 3).wait()
                load_desc(u, u % 3).start()

        @pl.when(slot_win(s) < LAST_WIN)
        def _(s=s, b=b):
            load_desc(s, b).wait()
            _process(slot_win(s), wins[b])
            store_desc(s, b).start()

        @pl.when(slot_win(s) == LAST_WIN)
        def _(s=s):
            row0 = LAST_WIN * WIN_R
            pltpu.sync_copy(q_hbm.at[pl.ds(row0, LAST_R)], wlast_v)
            _process(jnp.int32(LAST_WIN), wlast_v)
            pltpu.sync_copy(wlast_v, out_hbm.at[pl.ds(row0, LAST_R)])

    for s in range(SLOTS):
        u = s + 3
        pend = slot_win(s) < LAST_WIN
        if u < SLOTS:
            pend = pend & (slot_win(u) >= LAST_WIN)

        @pl.when(pend)
        def _(s=s):
            store_desc(s, s % 3).wait()


_apply_call = functools.partial(
    pl.kernel,
    out_type=jax.ShapeDtypeStruct((NSTATES, NACT), jnp.float32),
    mesh=_mesh,
    scratch_types=[
        pltpu.VMEM((KCH,), jnp.int32),
        pltpu.VMEM((KCH,), jnp.int32),
        pltpu.VMEM((KCH,), jnp.float32),
        pltpu.VMEM((KCH,), jnp.float32),
        pltpu.VMEM((CAP,), jnp.int32),
        pltpu.VMEM((CAP,), jnp.float32),
        pltpu.VMEM((WIN_R, NACT), jnp.float32),
        pltpu.VMEM((WIN_R, NACT), jnp.float32),
        pltpu.VMEM((WIN_R, NACT), jnp.float32),
        pltpu.VMEM((LAST_R, NACT), jnp.float32),
        pltpu.VMEM((CAP,), jnp.int32),
        pltpu.VMEM((CAP,), jnp.float32),
        pltpu.SemaphoreType.DMA,
        pltpu.SemaphoreType.DMA,
        pltpu.SemaphoreType.DMA,
        pltpu.SemaphoreType.DMA,
        pltpu.SemaphoreType.DMA,
        pltpu.SemaphoreType.DMA,
        pltpu.SemaphoreType.DMA,
        pltpu.SemaphoreType.DMA,
    ],
    compiler_params=_params,
    name="q_apply",
)(_apply_body)


def kernel(q_table, states, actions, rewards, next_states, dones):
    states = states.astype(jnp.int32)
    actions = actions.astype(jnp.int32)
    next_states = next_states.astype(jnp.int32)
    dones_f = dones.astype(jnp.float32)
    keys, tgts = _targets_call(q_table, next_states, states, actions,
                               rewards, dones_f)
    return _apply_call(q_table, keys, tgts)
